# baseline (speedup 1.0000x reference)
import functools

import jax
import jax.numpy as jnp
from jax import lax
from jax.experimental import pallas as pl
from jax.experimental.pallas import tpu as pltpu


def _conv_image_fast(w_ref, x, W, pad, mneg, mpos):
    Cin, HW = x.shape
    LA = HW + 128
    zl = jnp.zeros((Cin, pad), jnp.bfloat16)
    zr = jnp.zeros((Cin, 192), jnp.bfloat16)
    xp = jnp.concatenate([zl, x, zr], axis=1)

    def aux(o, mask):
        a = lax.slice(xp, (0, o), (Cin, o + LA))
        return a * mask if mask is not None else a

    a_m = aux(W - 1, mneg)
    a_c = aux(W, None)
    a_p = aux(W + 1, mpos)
    b_m = aux(2 * W - 1, mneg)
    b_p = aux(2 * W + 1, mpos)
    taps = [
        (a_m, 0), (a_c, 0), (a_p, 0),
        (b_m, 0), (xp, 2 * W), (b_p, 0),
        (a_m, 128), (a_c, 128), (a_p, 128),
    ]
    y = jnp.zeros((w_ref.shape[1], HW), jnp.float32)
    for k, (arr, off) in enumerate(taps):
        y = y + jnp.dot(w_ref[k], lax.slice(arr, (0, off), (Cin, off + HW)),
                        preferred_element_type=jnp.float32)
    return y


def _fused_kernel(w_ref, gamma_ref, beta_ref, x_ref, o_ref,
                  y_ref, sum_ref, ssq_ref, scale_ref, shift_ref,
                  *, NB, ipb, W, pad, M, eps):
    t = pl.program_id(0)

    @pl.when(t == 0)
    def _():
        sum_ref[...] = jnp.zeros_like(sum_ref)
        ssq_ref[...] = jnp.zeros_like(ssq_ref)

    @pl.when(t < NB)
    def _():
        HW = x_ref.shape[2]
        lane = lax.broadcasted_iota(jnp.int32, (1, HW + 128), 1) % W
        mneg = (lane != 0).astype(jnp.bfloat16)
        mpos = (lane != W - 1).astype(jnp.bfloat16)
        acc = jnp.zeros_like(sum_ref)
        ssq = jnp.zeros_like(ssq_ref)
        for j in range(ipb):
            y = _conv_image_fast(w_ref, x_ref[j].astype(jnp.bfloat16),
                                 W, pad, mneg, mpos)
            y_ref[t * ipb + j] = y
            acc += jnp.sum(y, axis=1, keepdims=True)
            ssq += jnp.sum(y * y, axis=1, keepdims=True)
        sum_ref[...] += acc
        ssq_ref[...] += ssq

    @pl.when(t == NB)
    def _():
        mean = sum_ref[...] * (1.0 / M)
        msq = ssq_ref[...] * (1.0 / M)
        var = jnp.maximum(msq - mean * mean, 0.0)
        scale = gamma_ref[...] * lax.rsqrt(var + eps)
        scale_ref[...] = scale
        shift_ref[...] = beta_ref[...] - mean * scale

    @pl.when(t >= NB)
    def _():
        i = t - NB
        scale = scale_ref[...]
        shift = shift_ref[...]
        for j in range(ipb):
            o_ref[j] = y_ref[i * ipb + j] * scale + shift


def kernel(x_nchw, w_oihw, gamma, beta):
    eps = 1e-5
    N, Cin, H, W = x_nchw.shape
    Cout, Cin_w, KH, KW = w_oihw.shape
    assert KH == 3 and KW == 3 and W % 64 == 0
    HW = H * W
    M = N * HW
    pad = 128

    x_flat = x_nchw.reshape(N, Cin, HW)
    wk = jnp.transpose(w_oihw, (2, 3, 0, 1)).reshape(KH * KW, Cout, Cin)
    wk = wk.astype(jnp.bfloat16)
    gamma_c = gamma.astype(jnp.float32).reshape(Cout, 1)
    beta_c = beta.astype(jnp.float32).reshape(Cout, 1)

    cparams = pltpu.CompilerParams(
        dimension_semantics=("arbitrary",),
        vmem_limit_bytes=63 * 1024 * 1024,
    )

    ipb = 4
    while N % ipb:
        ipb //= 2
    NB = N // ipb

    out = pl.pallas_call(
        functools.partial(_fused_kernel, NB=NB, ipb=ipb, W=W,
                          pad=pad, M=float(M), eps=eps),
        out_shape=jax.ShapeDtypeStruct((N, Cout, HW), jnp.float32),
        grid=(2 * NB,),
        in_specs=[
            pl.BlockSpec((KH * KW, Cout, Cin), lambda t: (0, 0, 0)),
            pl.BlockSpec((Cout, 1), lambda t: (0, 0)),
            pl.BlockSpec((Cout, 1), lambda t: (0, 0)),
            pl.BlockSpec((ipb, Cin, HW),
                         lambda t: (jnp.minimum(t, NB - 1), 0, 0)),
        ],
        out_specs=pl.BlockSpec((ipb, Cout, HW),
                               lambda t: (jnp.maximum(t - NB, 0), 0, 0)),
        scratch_shapes=[
            pltpu.VMEM((N, Cout, HW), jnp.float32),
            pltpu.VMEM((Cout, 1), jnp.float32),
            pltpu.VMEM((Cout, 1), jnp.float32),
            pltpu.VMEM((Cout, 1), jnp.float32),
            pltpu.VMEM((Cout, 1), jnp.float32),
        ],
        compiler_params=cparams,
    )(wk, gamma_c, beta_c, x_flat)

    return out.reshape(N, Cout, H, W)


# speedup vs baseline: 1.0002x; 1.0002x over previous
import functools

import jax
import jax.numpy as jnp
from jax import lax
from jax.experimental import pallas as pl
from jax.experimental.pallas import tpu as pltpu


def _conv_image_fast(w_ref, x, W, pad, mneg, mpos):
    Cin, HW = x.shape
    LA = HW + 128
    zl = jnp.zeros((Cin, pad), jnp.bfloat16)
    zr = jnp.zeros((Cin, 192), jnp.bfloat16)
    xp = jnp.concatenate([zl, x, zr], axis=1)

    def aux(o, mask):
        a = lax.slice(xp, (0, o), (Cin, o + LA))
        return a * mask if mask is not None else a

    a_m = aux(W - 1, mneg)
    a_c = aux(W, None)
    a_p = aux(W + 1, mpos)
    b_m = aux(2 * W - 1, mneg)
    b_p = aux(2 * W + 1, mpos)
    taps = [
        (a_m, 0), (a_c, 0), (a_p, 0),
        (b_m, 0), (xp, 2 * W), (b_p, 0),
        (a_m, 128), (a_c, 128), (a_p, 128),
    ]
    y = jnp.zeros((w_ref.shape[1], HW), jnp.float32)
    for k, (arr, off) in enumerate(taps):
        y = y + jnp.dot(w_ref[k], lax.slice(arr, (0, off), (Cin, off + HW)),
                        preferred_element_type=jnp.float32)
    return y


def _fused_kernel(w_ref, gamma_ref, beta_ref, x_ref, o_ref,
                  y_ref, sum_ref, ssq_ref, scale_ref, shift_ref,
                  *, NB, ipb, W, pad, M, eps):
    t = pl.program_id(0)

    @pl.when(t == 0)
    def _():
        sum_ref[...] = jnp.zeros_like(sum_ref)
        ssq_ref[...] = jnp.zeros_like(ssq_ref)

    @pl.when(t < NB)
    def _():
        HW = x_ref.shape[2]
        lane = lax.broadcasted_iota(jnp.int32, (1, HW + 128), 1) % W
        mneg = (lane != 0).astype(jnp.bfloat16)
        mpos = (lane != W - 1).astype(jnp.bfloat16)
        acc = jnp.zeros_like(sum_ref)
        ssq = jnp.zeros_like(ssq_ref)
        for j in range(ipb):
            y = _conv_image_fast(w_ref, x_ref[j].astype(jnp.bfloat16),
                                 W, pad, mneg, mpos)
            y_ref[t * ipb + j] = y.astype(jnp.bfloat16)
            acc += jnp.sum(y, axis=1, keepdims=True)
            ssq += jnp.sum(y * y, axis=1, keepdims=True)
        sum_ref[...] += acc
        ssq_ref[...] += ssq

    @pl.when(t == NB)
    def _():
        mean = sum_ref[...] * (1.0 / M)
        msq = ssq_ref[...] * (1.0 / M)
        var = jnp.maximum(msq - mean * mean, 0.0)
        scale = gamma_ref[...] * lax.rsqrt(var + eps)
        scale_ref[...] = scale
        shift_ref[...] = beta_ref[...] - mean * scale

    @pl.when(t >= NB)
    def _():
        i = t - NB
        scale = scale_ref[...]
        shift = shift_ref[...]
        for j in range(ipb):
            o_ref[j] = y_ref[i * ipb + j].astype(jnp.float32) * scale + shift


def kernel(x_nchw, w_oihw, gamma, beta):
    eps = 1e-5
    N, Cin, H, W = x_nchw.shape
    Cout, Cin_w, KH, KW = w_oihw.shape
    assert KH == 3 and KW == 3 and W % 64 == 0
    HW = H * W
    M = N * HW
    pad = 128

    x_flat = x_nchw.reshape(N, Cin, HW)
    wk = jnp.transpose(w_oihw, (2, 3, 0, 1)).reshape(KH * KW, Cout, Cin)
    wk = wk.astype(jnp.bfloat16)
    gamma_c = gamma.astype(jnp.float32).reshape(Cout, 1)
    beta_c = beta.astype(jnp.float32).reshape(Cout, 1)

    cparams = pltpu.CompilerParams(
        dimension_semantics=("arbitrary",),
        vmem_limit_bytes=63 * 1024 * 1024,
    )

    ipb = 4
    while N % ipb:
        ipb //= 2
    NB = N // ipb

    out = pl.pallas_call(
        functools.partial(_fused_kernel, NB=NB, ipb=ipb, W=W,
                          pad=pad, M=float(M), eps=eps),
        out_shape=jax.ShapeDtypeStruct((N, Cout, HW), jnp.float32),
        grid=(2 * NB,),
        in_specs=[
            pl.BlockSpec((KH * KW, Cout, Cin), lambda t: (0, 0, 0)),
            pl.BlockSpec((Cout, 1), lambda t: (0, 0)),
            pl.BlockSpec((Cout, 1), lambda t: (0, 0)),
            pl.BlockSpec((ipb, Cin, HW),
                         lambda t: (jnp.minimum(t, NB - 1), 0, 0)),
        ],
        out_specs=pl.BlockSpec((ipb, Cout, HW),
                               lambda t: (jnp.maximum(t - NB, 0), 0, 0)),
        scratch_shapes=[
            pltpu.VMEM((N, Cout, HW), jnp.bfloat16),
            pltpu.VMEM((Cout, 1), jnp.float32),
            pltpu.VMEM((Cout, 1), jnp.float32),
            pltpu.VMEM((Cout, 1), jnp.float32),
            pltpu.VMEM((Cout, 1), jnp.float32),
        ],
        compiler_params=cparams,
    )(wk, gamma_c, beta_c, x_flat)

    return out.reshape(N, Cout, H, W)


# speedup vs baseline: 1.0080x; 1.0078x over previous
import functools

import jax
import jax.numpy as jnp
from jax import lax
from jax.experimental import pallas as pl
from jax.experimental.pallas import tpu as pltpu


def _conv_image_fast(w_ref, x, W, pad, mneg, mpos):
    Cin, HW = x.shape
    LA = HW + 128
    zl = jnp.zeros((Cin, pad), jnp.bfloat16)
    zr = jnp.zeros((Cin, 192), jnp.bfloat16)
    xp = jnp.concatenate([zl, x, zr], axis=1)

    def aux(o, mask):
        a = lax.slice(xp, (0, o), (Cin, o + LA))
        return a * mask if mask is not None else a

    a_m = aux(W - 1, mneg)
    a_c = aux(W, None)
    a_p = aux(W + 1, mpos)
    b_m = aux(2 * W - 1, mneg)
    b_p = aux(2 * W + 1, mpos)
    taps = [
        (a_m, 0), (a_c, 0), (a_p, 0),
        (b_m, 0), (xp, 2 * W), (b_p, 0),
        (a_m, 128), (a_c, 128), (a_p, 128),
    ]
    y = jnp.zeros((w_ref.shape[1], HW), jnp.float32)
    for k, (arr, off) in enumerate(taps):
        y = y + jnp.dot(w_ref[k], lax.slice(arr, (0, off), (Cin, off + HW)),
                        preferred_element_type=jnp.float32)
    return y


def _fused_kernel(w_ref, gamma_ref, beta_ref, x_ref, o_ref,
                  y_ref, sum_ref, ssq_ref, scale_ref, shift_ref,
                  *, NB, ipb, W, pad, M, eps):
    t = pl.program_id(0)

    @pl.when(t == 0)
    def _():
        sum_ref[...] = jnp.zeros_like(sum_ref)
        ssq_ref[...] = jnp.zeros_like(ssq_ref)

    @pl.when(t < NB)
    def _():
        HW = x_ref.shape[2]
        lane = lax.broadcasted_iota(jnp.int32, (1, HW + 128), 1) % W
        mneg = (lane != 0).astype(jnp.bfloat16)
        mpos = (lane != W - 1).astype(jnp.bfloat16)
        acc = jnp.zeros_like(sum_ref)
        ssq = jnp.zeros_like(ssq_ref)
        for j in range(ipb):
            y = _conv_image_fast(w_ref, x_ref[j].astype(jnp.bfloat16),
                                 W, pad, mneg, mpos)
            y_ref[t * ipb + j] = y.astype(jnp.bfloat16)
            acc += jnp.sum(y, axis=1, keepdims=True)
            ssq += jnp.sum(y * y, axis=1, keepdims=True)
        sum_ref[...] += acc
        ssq_ref[...] += ssq

    @pl.when(t == NB)
    def _():
        mean = sum_ref[...] * (1.0 / M)
        msq = ssq_ref[...] * (1.0 / M)
        var = jnp.maximum(msq - mean * mean, 0.0)
        scale = gamma_ref[...] * lax.rsqrt(var + eps)
        scale_ref[...] = scale
        shift_ref[...] = beta_ref[...] - mean * scale

    @pl.when(t >= NB)
    def _():
        i = t - NB
        scale = scale_ref[...]
        shift = shift_ref[...]
        for j in range(ipb):
            o_ref[j] = y_ref[i * ipb + j].astype(jnp.float32) * scale + shift


def kernel(x_nchw, w_oihw, gamma, beta):
    eps = 1e-5
    N, Cin, H, W = x_nchw.shape
    Cout, Cin_w, KH, KW = w_oihw.shape
    assert KH == 3 and KW == 3 and W % 64 == 0
    HW = H * W
    M = N * HW
    pad = 128

    x_flat = x_nchw.reshape(N, Cin, HW)
    wk = jnp.transpose(w_oihw, (2, 3, 0, 1)).reshape(KH * KW, Cout, Cin)
    wk = wk.astype(jnp.bfloat16)
    gamma_c = gamma.astype(jnp.float32).reshape(Cout, 1)
    beta_c = beta.astype(jnp.float32).reshape(Cout, 1)

    cparams = pltpu.CompilerParams(
        dimension_semantics=("arbitrary",),
        vmem_limit_bytes=63 * 1024 * 1024,
    )

    ipb = 8
    while N % ipb:
        ipb //= 2
    NB = N // ipb

    out = pl.pallas_call(
        functools.partial(_fused_kernel, NB=NB, ipb=ipb, W=W,
                          pad=pad, M=float(M), eps=eps),
        out_shape=jax.ShapeDtypeStruct((N, Cout, HW), jnp.float32),
        grid=(2 * NB,),
        in_specs=[
            pl.BlockSpec((KH * KW, Cout, Cin), lambda t: (0, 0, 0)),
            pl.BlockSpec((Cout, 1), lambda t: (0, 0)),
            pl.BlockSpec((Cout, 1), lambda t: (0, 0)),
            pl.BlockSpec((ipb, Cin, HW),
                         lambda t: (jnp.minimum(t, NB - 1), 0, 0)),
        ],
        out_specs=pl.BlockSpec((ipb, Cout, HW),
                               lambda t: (jnp.maximum(t - NB, 0), 0, 0)),
        scratch_shapes=[
            pltpu.VMEM((N, Cout, HW), jnp.bfloat16),
            pltpu.VMEM((Cout, 1), jnp.float32),
            pltpu.VMEM((Cout, 1), jnp.float32),
            pltpu.VMEM((Cout, 1), jnp.float32),
            pltpu.VMEM((Cout, 1), jnp.float32),
        ],
        compiler_params=cparams,
    )(wk, gamma_c, beta_c, x_flat)

    return out.reshape(N, Cout, H, W)


# speedup vs baseline: 1.0228x; 1.0146x over previous
import functools

import jax
import jax.numpy as jnp
from jax import lax
from jax.experimental import pallas as pl
from jax.experimental.pallas import tpu as pltpu


def _conv_image_fast(w_ref, x, W, pad, mneg, mpos):
    Cin, HW = x.shape
    LA = HW + 128
    zl = jnp.zeros((Cin, pad), jnp.bfloat16)
    zr = jnp.zeros((Cin, 192), jnp.bfloat16)
    xp = jnp.concatenate([zl, x, zr], axis=1)
    xpm = xp * mneg
    xpp = xp * mpos

    a_m = lax.slice(xpm, (0, W - 1), (Cin, W - 1 + LA))
    a_c = lax.slice(xp, (0, W), (Cin, W + LA))
    a_p = lax.slice(xpp, (0, W + 1), (Cin, W + 1 + LA))
    b_m = lax.slice(xpm, (0, 2 * W - 1), (Cin, 2 * W - 1 + LA))
    b_p = lax.slice(xpp, (0, 2 * W + 1), (Cin, 2 * W + 1 + LA))
    taps = [
        (a_m, 0), (a_c, 0), (a_p, 0),
        (b_m, 0), (xp, 2 * W), (b_p, 0),
        (a_m, 128), (a_c, 128), (a_p, 128),
    ]
    y = jnp.zeros((w_ref.shape[1], HW), jnp.float32)
    for k, (arr, off) in enumerate(taps):
        y = y + jnp.dot(w_ref[k], lax.slice(arr, (0, off), (Cin, off + HW)),
                        preferred_element_type=jnp.float32)
    return y


def _fused_kernel(w_ref, gamma_ref, beta_ref, x_ref, o_ref,
                  y_ref, sum_ref, ssq_ref, scale_ref, shift_ref,
                  *, NB, ipb, W, pad, M, eps):
    t = pl.program_id(0)

    @pl.when(t == 0)
    def _():
        sum_ref[...] = jnp.zeros_like(sum_ref)
        ssq_ref[...] = jnp.zeros_like(ssq_ref)

    @pl.when(t < NB)
    def _():
        HW = x_ref.shape[2]
        lane = lax.broadcasted_iota(jnp.int32, (1, HW + 320), 1) % W
        mneg = (lane != W - 1).astype(jnp.bfloat16)
        mpos = (lane != 0).astype(jnp.bfloat16)
        acc = jnp.zeros_like(sum_ref)
        ssq = jnp.zeros_like(ssq_ref)
        for j in range(ipb):
            y = _conv_image_fast(w_ref, x_ref[j].astype(jnp.bfloat16),
                                 W, pad, mneg, mpos)
            y_ref[t * ipb + j] = y.astype(jnp.bfloat16)
            acc += jnp.sum(y, axis=1, keepdims=True)
            ssq += jnp.sum(y * y, axis=1, keepdims=True)
        sum_ref[...] += acc
        ssq_ref[...] += ssq

    @pl.when(t == NB)
    def _():
        mean = sum_ref[...] * (1.0 / M)
        msq = ssq_ref[...] * (1.0 / M)
        var = jnp.maximum(msq - mean * mean, 0.0)
        scale = gamma_ref[...] * lax.rsqrt(var + eps)
        scale_ref[...] = scale
        shift_ref[...] = beta_ref[...] - mean * scale

    @pl.when(t >= NB)
    def _():
        i = t - NB
        scale = scale_ref[...]
        shift = shift_ref[...]
        for j in range(ipb):
            o_ref[j] = y_ref[i * ipb + j].astype(jnp.float32) * scale + shift


def kernel(x_nchw, w_oihw, gamma, beta):
    eps = 1e-5
    N, Cin, H, W = x_nchw.shape
    Cout, Cin_w, KH, KW = w_oihw.shape
    assert KH == 3 and KW == 3 and W % 64 == 0
    HW = H * W
    M = N * HW
    pad = 128

    x_flat = x_nchw.reshape(N, Cin, HW)
    wk = jnp.transpose(w_oihw, (2, 3, 0, 1)).reshape(KH * KW, Cout, Cin)
    wk = wk.astype(jnp.bfloat16)
    gamma_c = gamma.astype(jnp.float32).reshape(Cout, 1)
    beta_c = beta.astype(jnp.float32).reshape(Cout, 1)

    cparams = pltpu.CompilerParams(
        dimension_semantics=("arbitrary",),
        vmem_limit_bytes=63 * 1024 * 1024,
    )

    ipb = 8
    while N % ipb:
        ipb //= 2
    NB = N // ipb

    out = pl.pallas_call(
        functools.partial(_fused_kernel, NB=NB, ipb=ipb, W=W,
                          pad=pad, M=float(M), eps=eps),
        out_shape=jax.ShapeDtypeStruct((N, Cout, HW), jnp.float32),
        grid=(2 * NB,),
        in_specs=[
            pl.BlockSpec((KH * KW, Cout, Cin), lambda t: (0, 0, 0)),
            pl.BlockSpec((Cout, 1), lambda t: (0, 0)),
            pl.BlockSpec((Cout, 1), lambda t: (0, 0)),
            pl.BlockSpec((ipb, Cin, HW),
                         lambda t: (jnp.minimum(t, NB - 1), 0, 0)),
        ],
        out_specs=pl.BlockSpec((ipb, Cout, HW),
                               lambda t: (jnp.maximum(t - NB, 0), 0, 0)),
        scratch_shapes=[
            pltpu.VMEM((N, Cout, HW), jnp.bfloat16),
            pltpu.VMEM((Cout, 1), jnp.float32),
            pltpu.VMEM((Cout, 1), jnp.float32),
            pltpu.VMEM((Cout, 1), jnp.float32),
            pltpu.VMEM((Cout, 1), jnp.float32),
        ],
        compiler_params=cparams,
    )(wk, gamma_c, beta_c, x_flat)

    return out.reshape(N, Cout, H, W)


# speedup vs baseline: 1.1070x; 1.0824x over previous
import functools

import jax
import jax.numpy as jnp
from jax import lax
from jax.experimental import pallas as pl
from jax.experimental.pallas import tpu as pltpu


def _conv_image_fast(w_ref, x, W, pad, mneg, mpos):
    Cin, HW = x.shape
    LA = HW + 128
    zl = jnp.zeros((Cin, pad), jnp.bfloat16)
    zr = jnp.zeros((Cin, 192), jnp.bfloat16)
    xp = jnp.concatenate([zl, x, zr], axis=1)
    xpm = xp * mneg
    xpp = xp * mpos

    a_m = lax.slice(xpm, (0, W - 1), (Cin, W - 1 + LA))
    a_c = lax.slice(xp, (0, W), (Cin, W + LA))
    a_p = lax.slice(xpp, (0, W + 1), (Cin, W + 1 + LA))
    b_m = lax.slice(xpm, (0, 2 * W - 1), (Cin, 2 * W - 1 + LA))
    b_p = lax.slice(xpp, (0, 2 * W + 1), (Cin, 2 * W + 1 + LA))

    def two(a):
        return jnp.concatenate(
            [lax.slice(a, (0, 0), (Cin, HW)),
             lax.slice(a, (0, 128), (Cin, 128 + HW))], axis=0)

    rhs = [
        two(a_m), two(a_c), two(a_p),
        lax.slice(b_m, (0, 0), (Cin, HW)),
        lax.slice(xp, (0, 2 * W), (Cin, 2 * W + HW)),
        lax.slice(b_p, (0, 0), (Cin, HW)),
    ]
    y = jnp.zeros((w_ref.shape[1], HW), jnp.float32)
    for k, r in enumerate(rhs):
        wk = w_ref[k] if k < 3 else w_ref[k][:, :Cin]
        y = y + jnp.dot(wk, r, preferred_element_type=jnp.float32)
    return y


def _fused_kernel(w_ref, gamma_ref, beta_ref, x_ref, o_ref,
                  y_ref, sum_ref, ssq_ref, scale_ref, shift_ref,
                  *, NB, ipb, W, pad, M, eps):
    t = pl.program_id(0)

    @pl.when(t == 0)
    def _():
        sum_ref[...] = jnp.zeros_like(sum_ref)
        ssq_ref[...] = jnp.zeros_like(ssq_ref)

    @pl.when(t < NB)
    def _():
        HW = x_ref.shape[2]
        lane = lax.broadcasted_iota(jnp.int32, (1, HW + 320), 1) % W
        mneg = (lane != W - 1).astype(jnp.bfloat16)
        mpos = (lane != 0).astype(jnp.bfloat16)
        acc = jnp.zeros_like(sum_ref)
        ssq = jnp.zeros_like(ssq_ref)
        for j in range(ipb):
            y = _conv_image_fast(w_ref, x_ref[j].astype(jnp.bfloat16),
                                 W, pad, mneg, mpos)
            y_ref[t * ipb + j] = y.astype(jnp.bfloat16)
            acc += jnp.sum(y, axis=1, keepdims=True)
            ssq += jnp.sum(y * y, axis=1, keepdims=True)
        sum_ref[...] += acc
        ssq_ref[...] += ssq

    @pl.when(t == NB)
    def _():
        mean = sum_ref[...] * (1.0 / M)
        msq = ssq_ref[...] * (1.0 / M)
        var = jnp.maximum(msq - mean * mean, 0.0)
        scale = gamma_ref[...] * lax.rsqrt(var + eps)
        scale_ref[...] = scale
        shift_ref[...] = beta_ref[...] - mean * scale

    @pl.when(t >= NB)
    def _():
        i = t - NB
        scale = scale_ref[...]
        shift = shift_ref[...]
        for j in range(ipb):
            o_ref[j] = y_ref[i * ipb + j].astype(jnp.float32) * scale + shift


def kernel(x_nchw, w_oihw, gamma, beta):
    eps = 1e-5
    N, Cin, H, W = x_nchw.shape
    Cout, Cin_w, KH, KW = w_oihw.shape
    assert KH == 3 and KW == 3 and W % 64 == 0
    HW = H * W
    M = N * HW
    pad = 128

    x_flat = x_nchw.reshape(N, Cin, HW)
    zpad = jnp.zeros((Cout, Cin), w_oihw.dtype)
    mats = ([jnp.concatenate([w_oihw[:, :, 0, kw], w_oihw[:, :, 2, kw]],
                             axis=1) for kw in range(KW)] +
            [jnp.concatenate([w_oihw[:, :, 1, kw], zpad], axis=1)
             for kw in range(KW)])
    wk = jnp.stack(mats, axis=0).astype(jnp.bfloat16)
    gamma_c = gamma.astype(jnp.float32).reshape(Cout, 1)
    beta_c = beta.astype(jnp.float32).reshape(Cout, 1)

    cparams = pltpu.CompilerParams(
        dimension_semantics=("arbitrary",),
        vmem_limit_bytes=63 * 1024 * 1024,
    )

    ipb = 8
    while N % ipb:
        ipb //= 2
    NB = N // ipb

    out = pl.pallas_call(
        functools.partial(_fused_kernel, NB=NB, ipb=ipb, W=W,
                          pad=pad, M=float(M), eps=eps),
        out_shape=jax.ShapeDtypeStruct((N, Cout, HW), jnp.float32),
        grid=(2 * NB,),
        in_specs=[
            pl.BlockSpec((2 * KW, Cout, 2 * Cin), lambda t: (0, 0, 0)),
            pl.BlockSpec((Cout, 1), lambda t: (0, 0)),
            pl.BlockSpec((Cout, 1), lambda t: (0, 0)),
            pl.BlockSpec((ipb, Cin, HW),
                         lambda t: (jnp.minimum(t, NB - 1), 0, 0)),
        ],
        out_specs=pl.BlockSpec((ipb, Cout, HW),
                               lambda t: (jnp.maximum(t - NB, 0), 0, 0)),
        scratch_shapes=[
            pltpu.VMEM((N, Cout, HW), jnp.bfloat16),
            pltpu.VMEM((Cout, 1), jnp.float32),
            pltpu.VMEM((Cout, 1), jnp.float32),
            pltpu.VMEM((Cout, 1), jnp.float32),
            pltpu.VMEM((Cout, 1), jnp.float32),
        ],
        compiler_params=cparams,
    )(wk, gamma_c, beta_c, x_flat)

    return out.reshape(N, Cout, H, W)
